# Initial kernel scaffold; baseline (speedup 1.0000x reference)
import jax
import jax.numpy as jnp
from jax.experimental import pallas as pl


def kernel(x, ptx, W_sm, b_sm, W_feat, b_feat, bs, height, width, point_key, pixel_tgt_idx):
    raise NotImplementedError("write your pallas kernel here")


# speedup vs baseline: 2.1581x; 2.1581x over previous
import jax
import jax.numpy as jnp
from jax.experimental import pallas as pl

_BP = 4096


def _pt_kernel(ptx_ref, wc_ref, clx_ref, a_ref, t_ref):
    sa = jnp.dot(ptx_ref[...], wc_ref[...], preferred_element_type=jnp.float32)
    e = jnp.exp(sa[:, 0:1])
    a = sa[:, 1:]
    a_ref[...] = a
    v = jnp.concatenate([e, e * a], axis=1)
    c = v
    off = 1
    rows = v.shape[0]
    while off < rows:
        c = c + jnp.concatenate(
            [jnp.zeros((off, c.shape[1]), c.dtype), c[:-off]], axis=0
        )
        off *= 2
    clx_ref[...] = c - v
    t_ref[pl.ds(pl.program_id(0), 1), :] = c[-1:, :]


def _px_kernel(f_ref, w_ref, b_ref, g_ref):
    g_ref[...] = (
        jnp.dot(f_ref[...], w_ref[...], preferred_element_type=jnp.float32)
        + b_ref[...]
    )


def kernel(x, ptx, W_sm, b_sm, W_feat, b_feat, bs, height, width, point_key, pixel_tgt_idx):
    n_points, ptxch = ptx.shape
    n_pix = point_key.shape[0] - 1
    _, xch, h, w = x.shape
    out_ch = W_feat.shape[0]
    nb = n_points // _BP

    wc = jnp.concatenate([W_sm[:, :ptxch].T, W_feat[:, :ptxch].T], axis=1)
    clx, a_pt, t = pl.pallas_call(
        _pt_kernel,
        grid=(nb,),
        in_specs=[
            pl.BlockSpec((_BP, ptxch), lambda i: (i, 0)),
            pl.BlockSpec((ptxch, ptxch + 1), lambda i: (0, 0)),
        ],
        out_specs=[
            pl.BlockSpec((_BP, ptxch + 1), lambda i: (i, 0)),
            pl.BlockSpec((_BP, ptxch), lambda i: (i, 0)),
            pl.BlockSpec((nb, ptxch + 1), lambda i: (0, 0)),
        ],
        out_shape=[
            jax.ShapeDtypeStruct((n_points, ptxch + 1), jnp.float32),
            jax.ShapeDtypeStruct((n_points, ptxch), jnp.float32),
            jax.ShapeDtypeStruct((nb, ptxch + 1), jnp.float32),
        ],
    )(ptx, wc)

    feat_pix = jnp.transpose(x, (0, 2, 3, 1)).reshape(h * w, xch)[pixel_tgt_idx]
    npb = n_pix // _BP
    g = pl.pallas_call(
        _px_kernel,
        grid=(npb,),
        in_specs=[
            pl.BlockSpec((_BP, xch), lambda i: (i, 0)),
            pl.BlockSpec((xch, out_ch), lambda i: (0, 0)),
            pl.BlockSpec((1, out_ch), lambda i: (0, 0)),
        ],
        out_specs=pl.BlockSpec((_BP, out_ch), lambda i: (i, 0)),
        out_shape=jax.ShapeDtypeStruct((n_pix, out_ch), jnp.float32),
    )(feat_pix, W_feat[:, ptxch:].T, b_feat.reshape(1, out_ch))

    p = jnp.concatenate(
        [jnp.zeros((1, ptxch + 1), jnp.float32), jnp.cumsum(t, axis=0)], axis=0
    )
    clxp = jnp.concatenate([clx, jnp.zeros((1, ptxch + 1), jnp.float32)], axis=0)
    pk = point_key.astype(jnp.int32)
    lo, hi = pk[:-1], pk[1:]
    d = (clxp[hi] - clxp[lo]) + (p[hi // _BP] - p[lo // _BP])
    E = d[:, 0:1]
    M = d[:, 1:]
    seg = (M + g * E) / (E + 1e-16)

    counts = hi - lo
    seg_ids = jnp.repeat(jnp.arange(n_pix), counts, total_repeat_length=n_points)
    ptx_out = a_pt + g[seg_ids]

    fmap = jnp.zeros((h * w, out_ch), jnp.float32).at[pixel_tgt_idx].set(seg)
    mask = jnp.zeros((h * w, 1), jnp.float32).at[pixel_tgt_idx].set(1.0)
    fmap = jnp.transpose(fmap.reshape(1, h, w, out_ch), (0, 3, 1, 2))
    mask = jnp.transpose(mask.reshape(1, h, w, 1), (0, 3, 1, 2))
    x_out = jnp.where(mask > 0, fmap, x)
    return (x_out, ptx_out)
